# baseline (speedup 1.0000x reference)
import functools

import jax
import jax.numpy as jnp
from jax import lax
from jax.experimental import pallas as pl
from jax.experimental.pallas import tpu as pltpu
from jax.experimental.pallas import tpu_sc as plsc

NC = 2
NS = 16
C = 128


def _sc_stage(table, expr2d, tok2d, sym2d, idtab, ididx3d,
              acc_rows, id_rows, d, max_tok):
  per_sub = acc_rows // NS
  wo_chunks = per_sub // C
  k_app = expr2d.shape[0] // (NC * NS)
  id_chunks = id_rows // C
  mesh = plsc.VectorSubcoreMesh(core_axis_name="c", subcore_axis_name="s")

  @functools.partial(
      pl.kernel,
      mesh=mesh,
      out_type=[jax.ShapeDtypeStruct((NC, acc_rows, d), jnp.float32),
                jax.ShapeDtypeStruct((NC, acc_rows, d), jnp.float32),
                jax.ShapeDtypeStruct((id_rows, d), jnp.float32)],
      scratch_types=[
          pltpu.VMEM((C, d), jnp.float32),
          pltpu.VMEM((C, d), jnp.float32),
          pltpu.VMEM_SHARED((acc_rows, d), jnp.float32),
          pltpu.VMEM((8, C), jnp.int32),
          pltpu.VMEM((8, C), jnp.int32),
          pltpu.VMEM((8, C), jnp.int32),
          pltpu.VMEM((1, C), jnp.int32),
          pltpu.SemaphoreType.DMA,
          pltpu.SemaphoreType.DMA,
      ],
  )
  def sc_kernel(table_hbm, expr_hbm, tok_hbm, sym_hbm, idtab_hbm, ididx_hbm,
                sums_out, counts_out, ids_out,
                rowbuf, rowbuf2, acc, ebuf, tbuf, sbuf, idixbuf,
                sem, sem2):
    ci = lax.axis_index("c")
    si = lax.axis_index("s")
    wid = ci * NS + si
    base = wid * k_app

    def _fill(val):
      def _f(i, _):
        r = i // (d // 16)
        c0 = (i % (d // 16)) * 16
        rowbuf[r, pl.ds(c0, 16)] = jnp.full((16,), val, jnp.float32)
        return 0
      lax.fori_loop(0, C * (d // 16), _f, 0)

    def _zero_acc():
      def _z(k, _):
        r0 = si * per_sub + k * C
        pltpu.sync_copy(rowbuf, acc.at[pl.ds(r0, C)])
        return 0
      lax.fori_loop(0, wo_chunks, _z, 0)

    def _writeout(dst):
      def _w(k, _):
        r0 = si * per_sub + k * C
        pltpu.sync_copy(acc.at[pl.ds(r0, C)], rowbuf)
        pltpu.sync_copy(rowbuf, dst.at[ci, pl.ds(r0, C)])
        return 0
      lax.fori_loop(0, wo_chunks, _w, 0)

    _fill(0.0)
    _zero_acc()

    plsc.subcore_barrier()

    def _blk(b, _):
      r0 = base + b * 8
      pltpu.sync_copy(expr_hbm.at[pl.ds(r0, 8)], ebuf)
      pltpu.sync_copy(tok_hbm.at[pl.ds(r0, 8)], tbuf)
      pltpu.sync_copy(sym_hbm.at[pl.ds(r0, 8)], sbuf)

      def _flat(i, _):
        r = i // (C // 16)
        c0 = (i % (C // 16)) * 16
        e = ebuf[r, pl.ds(c0, 16)]
        t = tbuf[r, pl.ds(c0, 16)]
        ebuf[r, pl.ds(c0, 16)] = e * max_tok + t
        return 0
      lax.fori_loop(0, 8 * (C // 16), _flat, 0)

      bufs = (rowbuf, rowbuf2)
      sms = (sem, sem2)
      cp = pltpu.async_copy(table_hbm.at[ebuf.at[0]], bufs[0], sms[0])
      for j in range(8):
        cp.wait()
        if j < 7:
          cp = pltpu.async_copy(table_hbm.at[ebuf.at[j + 1]],
                                bufs[(j + 1) % 2], sms[(j + 1) % 2])
        pltpu.sync_copy(bufs[j % 2], acc.at[sbuf.at[j]], add=True)
      return 0
    lax.fori_loop(0, k_app // 8, _blk, 0)

    for j in range((id_chunks + NC * NS - 1) // (NC * NS)):
      cid = wid + NC * NS * j

      @pl.when(cid < id_chunks)
      def _():
        pltpu.sync_copy(ididx_hbm.at[cid], idixbuf)
        pltpu.async_copy(idtab_hbm.at[idixbuf.at[0]], rowbuf, sem).wait()
        pltpu.sync_copy(rowbuf, ids_out.at[pl.ds(cid * C, C)])

    plsc.subcore_barrier()
    _writeout(sums_out)
    plsc.subcore_barrier()

    _fill(0.0)
    _zero_acc()
    plsc.subcore_barrier()
    _fill(1.0)

    def _blk2(b, _):
      r0 = base + b * 8
      pltpu.sync_copy(sym_hbm.at[pl.ds(r0, 8)], sbuf)

      def _chunk2(j, _):
        pltpu.sync_copy(rowbuf, acc.at[sbuf.at[j]], add=True)
        return 0
      lax.fori_loop(0, 8, _chunk2, 0)
      return 0
    lax.fori_loop(0, k_app // 8, _blk2, 0)

    plsc.subcore_barrier()
    _writeout(counts_out)

  return sc_kernel(table, expr2d, tok2d, sym2d, idtab, ididx3d)


def _tc_combine(sums, counts, ids, w, n_sym, d, r_blk):
  def body(sums_ref, cnts_ref, ids_ref, w_ref, out_ref):
    s = sums_ref[...]
    c = cnts_ref[...]
    idrows = ids_ref[...]
    wm = w_ref[...]
    cnt = jnp.maximum(c[0, :, 0:1] + c[1, :, 0:1], 1.0)
    mean = (s[0] + s[1]) / cnt
    out = lax.dot_general(idrows, wm[:, :d], (((1,), (1,)), ((), ())),
                          preferred_element_type=jnp.float32)
    out = out + lax.dot_general(mean, wm[:, d:], (((1,), (1,)), ((), ())),
                                preferred_element_type=jnp.float32)
    out_ref[...] = jnp.maximum(out, 0.0)

  return pl.pallas_call(
      body,
      grid=(n_sym // r_blk,),

      in_specs=[
          pl.BlockSpec((NC, r_blk, d), lambda i: (0, i, 0)),
          pl.BlockSpec((NC, r_blk, d), lambda i: (0, i, 0)),
          pl.BlockSpec((r_blk, d), lambda i: (i, 0)),
          pl.BlockSpec((d, 2 * d), lambda i: (0, 0)),
      ],
      out_specs=pl.BlockSpec((r_blk, d), lambda i: (i, 0)),
      out_shape=jax.ShapeDtypeStruct((n_sym, d), jnp.float32),
  )(sums, counts, ids, w)


def kernel(encoded_identifiers, symbols_identifier_indices,
           symbols_appearances_cfg_expression_idx,
           symbols_appearances_expression_token_idx,
           symbols_appearances_symbol_idx, encoded_cfg_expressions, W):
  d = encoded_identifiers.shape[1]
  n_sym = symbols_identifier_indices.shape[0]

  mt = encoded_cfg_expressions.shape[1]
  n_app = symbols_appearances_symbol_idx.shape[0]
  k_app = 80
  pad_app = NC * NS * k_app * C
  ei = jnp.zeros((pad_app,), jnp.int32).at[:n_app].set(
      symbols_appearances_cfg_expression_idx.astype(jnp.int32))
  ti = jnp.zeros((pad_app,), jnp.int32).at[:n_app].set(
      symbols_appearances_expression_token_idx.astype(jnp.int32))
  si = jnp.full((pad_app,), n_sym, jnp.int32).at[:n_app].set(
      symbols_appearances_symbol_idx.astype(jnp.int32))
  flat_expr = encoded_cfg_expressions.reshape(-1, d)

  acc_rows = 10240
  id_rows = acc_rows
  ii = jnp.zeros((id_rows,), jnp.int32).at[:n_sym].set(
      symbols_identifier_indices.astype(jnp.int32))

  sums, counts, ids = _sc_stage(flat_expr, ei.reshape(-1, C),
                               ti.reshape(-1, C), si.reshape(-1, C),
                               encoded_identifiers, ii.reshape(-1, 1, C),
                               acc_rows, id_rows, d, mt)

  out = _tc_combine(sums, counts, ids, W, acc_rows, d, 1024)
  return out[:n_sym]


# speedup vs baseline: 1.1338x; 1.1338x over previous
import functools

import jax
import jax.numpy as jnp
from jax import lax
from jax.experimental import pallas as pl
from jax.experimental.pallas import tpu as pltpu
from jax.experimental.pallas import tpu_sc as plsc

NC = 2
NS = 16
C = 128


def _sc_stage(table, expr2d, tok2d, sym2d, idtab, ididx3d,
              acc_rows, id_rows, d, max_tok):
  per_sub = acc_rows // NS
  wo_chunks = per_sub // C
  k_app = expr2d.shape[0] // (NC * NS)
  id_chunks = id_rows // C
  mesh = plsc.VectorSubcoreMesh(core_axis_name="c", subcore_axis_name="s")

  @functools.partial(
      pl.kernel,
      mesh=mesh,
      out_type=[jax.ShapeDtypeStruct((NC, acc_rows, d), jnp.float32),
                jax.ShapeDtypeStruct((NC, acc_rows, d), jnp.float32),
                jax.ShapeDtypeStruct((id_rows, d), jnp.float32)],
      scratch_types=[
          pltpu.VMEM((C, d), jnp.float32),
          pltpu.VMEM((C, d), jnp.float32),
          pltpu.VMEM_SHARED((acc_rows, d), jnp.float32),
          pltpu.VMEM((8, C), jnp.int32),
          pltpu.VMEM((8, C), jnp.int32),
          pltpu.VMEM((8, C), jnp.int32),
          pltpu.VMEM((1, C), jnp.int32),
          pltpu.SemaphoreType.DMA,
          pltpu.SemaphoreType.DMA,
      ],
  )
  def sc_kernel(table_hbm, expr_hbm, tok_hbm, sym_hbm, idtab_hbm, ididx_hbm,
                sums_out, counts_out, ids_out,
                rowbuf, rowbuf2, acc, ebuf, tbuf, sbuf, idixbuf,
                sem, sem2):
    ci = lax.axis_index("c")
    si = lax.axis_index("s")
    wid = ci * NS + si
    base = wid * k_app

    def _fill(val):
      def _f(i, _):
        r = i // (d // 16)
        c0 = (i % (d // 16)) * 16
        rowbuf[r, pl.ds(c0, 16)] = jnp.full((16,), val, jnp.float32)
        return 0
      lax.fori_loop(0, C * (d // 16), _f, 0)

    def _zero_acc():
      def _z(k, _):
        r0 = si * per_sub + k * C
        pltpu.sync_copy(rowbuf, acc.at[pl.ds(r0, C)])
        return 0
      lax.fori_loop(0, wo_chunks, _z, 0)

    def _writeout(dst):
      def _w(k, _):
        r0 = si * per_sub + k * C
        pltpu.sync_copy(acc.at[pl.ds(r0, C)], rowbuf)
        pltpu.sync_copy(rowbuf, dst.at[ci, pl.ds(r0, C)])
        return 0
      lax.fori_loop(0, wo_chunks, _w, 0)

    _fill(0.0)
    _zero_acc()

    plsc.subcore_barrier()

    def _blk(b, _):
      r0 = base + b * 8
      pltpu.sync_copy(expr_hbm.at[pl.ds(r0, 8)], ebuf)
      pltpu.sync_copy(tok_hbm.at[pl.ds(r0, 8)], tbuf)
      pltpu.sync_copy(sym_hbm.at[pl.ds(r0, 8)], sbuf)

      def _flat(i, _):
        r = i // (C // 16)
        c0 = (i % (C // 16)) * 16
        e = ebuf[r, pl.ds(c0, 16)]
        t = tbuf[r, pl.ds(c0, 16)]
        ebuf[r, pl.ds(c0, 16)] = e * max_tok + t
        return 0
      lax.fori_loop(0, 8 * (C // 16), _flat, 0)

      bufs = (rowbuf, rowbuf2)
      sms = (sem, sem2)
      cp = pltpu.async_copy(table_hbm.at[ebuf.at[0]], bufs[0], sms[0])
      for j in range(8):
        cp.wait()
        if j < 7:
          cp = pltpu.async_copy(table_hbm.at[ebuf.at[j + 1]],
                                bufs[(j + 1) % 2], sms[(j + 1) % 2])
        pltpu.sync_copy(bufs[j % 2], acc.at[sbuf.at[j]], add=True)
      return 0
    lax.fori_loop(0, k_app // 8, _blk, 0)

    for j in range((id_chunks + NC * NS - 1) // (NC * NS)):
      cid = wid + NC * NS * j

      @pl.when(cid < id_chunks)
      def _():
        pltpu.sync_copy(ididx_hbm.at[cid], idixbuf)
        pltpu.async_copy(idtab_hbm.at[idixbuf.at[0]], rowbuf, sem).wait()
        pltpu.sync_copy(rowbuf, ids_out.at[pl.ds(cid * C, C)])

    plsc.subcore_barrier()
    _writeout(sums_out)
    plsc.subcore_barrier()

    _fill(0.0)
    _zero_acc()
    plsc.subcore_barrier()
    _fill(1.0)

    def _blk2(b, _):
      r0 = base + b * 8
      pltpu.sync_copy(sym_hbm.at[pl.ds(r0, 8)], sbuf)

      def _chunk2(j, _):
        pltpu.sync_copy(rowbuf, acc.at[sbuf.at[j]], add=True)
        return 0
      lax.fori_loop(0, 8, _chunk2, 0)
      return 0
    lax.fori_loop(0, k_app // 8, _blk2, 0)

    plsc.subcore_barrier()
    _writeout(counts_out)

  return sc_kernel(table, expr2d, tok2d, sym2d, idtab, ididx3d)


def _tc_combine(sums, counts, ids, w, n_sym, d, r_blk):
  def body(sums_ref, cnts_ref, ids_ref, w_ref, out_ref):
    s = sums_ref[...]
    c = cnts_ref[...]
    idrows = ids_ref[...]
    wm = w_ref[...]
    cnt = jnp.maximum(c[0, :, 0:1] + c[1, :, 0:1], 1.0)
    mean = (s[0] + s[1]) / cnt
    out = lax.dot_general(idrows, wm[:, :d], (((1,), (1,)), ((), ())),
                          preferred_element_type=jnp.float32)
    out = out + lax.dot_general(mean, wm[:, d:], (((1,), (1,)), ((), ())),
                                preferred_element_type=jnp.float32)
    out_ref[...] = jnp.maximum(out, 0.0)

  return pl.pallas_call(
      body,
      grid=(n_sym // r_blk,),

      in_specs=[
          pl.BlockSpec((NC, r_blk, d), lambda i: (0, i, 0)),
          pl.BlockSpec((NC, r_blk, d), lambda i: (0, i, 0)),
          pl.BlockSpec((r_blk, d), lambda i: (i, 0)),
          pl.BlockSpec((d, 2 * d), lambda i: (0, 0)),
      ],
      out_specs=pl.BlockSpec((r_blk, d), lambda i: (i, 0)),
      out_shape=jax.ShapeDtypeStruct((n_sym, d), jnp.float32),
  )(sums, counts, ids, w)


def kernel(encoded_identifiers, symbols_identifier_indices,
           symbols_appearances_cfg_expression_idx,
           symbols_appearances_expression_token_idx,
           symbols_appearances_symbol_idx, encoded_cfg_expressions, W):
  d = encoded_identifiers.shape[1]
  n_sym = symbols_identifier_indices.shape[0]

  mt = encoded_cfg_expressions.shape[1]
  n_app = symbols_appearances_symbol_idx.shape[0]
  k_app = 80
  pad_app = NC * NS * k_app * C
  ei = jnp.zeros((pad_app,), jnp.int32).at[:n_app].set(
      symbols_appearances_cfg_expression_idx.astype(jnp.int32))
  ti = jnp.zeros((pad_app,), jnp.int32).at[:n_app].set(
      symbols_appearances_expression_token_idx.astype(jnp.int32))
  si = (n_sym + jnp.arange(pad_app, dtype=jnp.int32) % 240).at[:n_app].set(
      symbols_appearances_symbol_idx.astype(jnp.int32))
  flat_expr = encoded_cfg_expressions.reshape(-1, d)

  acc_rows = 10240
  id_rows = acc_rows
  ii = jnp.zeros((id_rows,), jnp.int32).at[:n_sym].set(
      symbols_identifier_indices.astype(jnp.int32))

  sums, counts, ids = _sc_stage(flat_expr, ei.reshape(-1, C),
                               ti.reshape(-1, C), si.reshape(-1, C),
                               encoded_identifiers, ii.reshape(-1, 1, C),
                               acc_rows, id_rows, d, mt)

  out = _tc_combine(sums, counts, ids, W, acc_rows, d, 1024)
  return out[:n_sym]
